# Initial kernel scaffold; baseline (speedup 1.0000x reference)
import jax
import jax.numpy as jnp
from jax.experimental import pallas as pl


def kernel(position_ids, table):
    raise NotImplementedError("write your pallas kernel here")


# speedup vs baseline: 1.2618x; 1.2618x over previous
import functools

import jax
import jax.numpy as jnp
from jax import lax
from jax.experimental import pallas as pl
from jax.experimental.pallas import tpu as pltpu
from jax.experimental.pallas import tpu_sc as plsc

HIDDEN = 1024
MAXPOS = 8192
BATCH = 4
SEQ = 8192

_info = plsc.get_sparse_core_info()
NC = _info.num_cores
NS = _info.num_subcores
NW = NC * NS
TOTAL = BATCH * SEQ
B_PER_W = TOTAL // NW
CHUNK = 64
NCHUNK = B_PER_W // CHUNK

_mesh = plsc.VectorSubcoreMesh(core_axis_name="c", subcore_axis_name="s")


@functools.partial(
    pl.kernel,
    mesh=_mesh,
    out_type=jax.ShapeDtypeStruct((TOTAL, HIDDEN), jnp.float32),
    scratch_types=[
        pltpu.VMEM((B_PER_W,), jnp.int32),
        pltpu.VMEM((CHUNK, HIDDEN), jnp.float32),
        pltpu.SemaphoreType.DMA,
    ],
)
def _emb_lookup(idx_hbm, table_hbm, out_hbm, idx_v, rows_v, sem):
    wid = lax.axis_index("s") * NC + lax.axis_index("c")
    base = wid * B_PER_W
    pltpu.sync_copy(idx_hbm.at[pl.ds(base, B_PER_W)], idx_v)

    def body(i, carry):
        off = i * CHUNK
        pltpu.async_copy(
            table_hbm.at[idx_v.at[pl.ds(off, CHUNK)]], rows_v, sem
        ).wait()
        pltpu.sync_copy(rows_v, out_hbm.at[pl.ds(base + off, CHUNK)])
        return carry

    lax.fori_loop(0, NCHUNK, body, 0)


def kernel(position_ids, table):
    idx_flat = jnp.swapaxes(position_ids, 0, 1).reshape(TOTAL).astype(jnp.int32)
    out = _emb_lookup(idx_flat, table)
    return out.reshape(SEQ, BATCH, HIDDEN)


# speedup vs baseline: 1.2808x; 1.0151x over previous
import functools

import jax
import jax.numpy as jnp
from jax import lax
from jax.experimental import pallas as pl
from jax.experimental.pallas import tpu as pltpu
from jax.experimental.pallas import tpu_sc as plsc

HIDDEN = 1024
MAXPOS = 8192
BATCH = 4
SEQ = 8192

_info = plsc.get_sparse_core_info()
NC = _info.num_cores
NS = _info.num_subcores
NW = NC * NS
TOTAL = BATCH * SEQ
B_PER_W = TOTAL // NW
CHUNK = 32
NCHUNK = B_PER_W // CHUNK
NPAIR = NCHUNK // 2

_mesh = plsc.VectorSubcoreMesh(core_axis_name="c", subcore_axis_name="s")


@functools.partial(
    pl.kernel,
    mesh=_mesh,
    out_type=jax.ShapeDtypeStruct((TOTAL, HIDDEN), jnp.float32),
    scratch_types=[
        pltpu.VMEM((B_PER_W,), jnp.int32),
        pltpu.VMEM((CHUNK, HIDDEN), jnp.float32),
        pltpu.VMEM((CHUNK, HIDDEN), jnp.float32),
        pltpu.SemaphoreType.DMA,
        pltpu.SemaphoreType.DMA,
        pltpu.SemaphoreType.DMA,
        pltpu.SemaphoreType.DMA,
    ],
)
def _emb_lookup(idx_hbm, table_hbm, out_hbm, idx_v, buf0, buf1,
                gsem0, gsem1, wsem0, wsem1):
    wid = lax.axis_index("s") * NC + lax.axis_index("c")
    base = wid * B_PER_W
    pltpu.sync_copy(idx_hbm.at[pl.ds(base, B_PER_W)], idx_v)

    def gather(chunk, buf, sem):
        return pltpu.make_async_copy(
            table_hbm.at[idx_v.at[pl.ds(chunk * CHUNK, CHUNK)]], buf, sem)

    def write(chunk, buf, sem):
        return pltpu.make_async_copy(
            buf, out_hbm.at[pl.ds(base + chunk * CHUNK, CHUNK)], sem)

    gather(0, buf0, gsem0).start()

    def body(g, carry):
        c0 = 2 * g
        gather(c0, buf0, gsem0).wait()
        write(c0, buf0, wsem0).start()
        pl.when(g > 0)(lambda: write(c0 - 1, buf1, wsem1).wait())
        gather(c0 + 1, buf1, gsem1).start()
        gather(c0 + 1, buf1, gsem1).wait()
        write(c0 + 1, buf1, wsem1).start()
        write(c0, buf0, wsem0).wait()
        pl.when(g < NPAIR - 1)(lambda: gather(c0 + 2, buf0, gsem0).start())
        return carry

    lax.fori_loop(0, NPAIR, body, 0)
    write(NCHUNK - 1, buf1, wsem1).wait()


def kernel(position_ids, table):
    idx_flat = jnp.swapaxes(position_ids, 0, 1).reshape(TOTAL).astype(jnp.int32)
    out = _emb_lookup(idx_flat, table)
    return out.reshape(SEQ, BATCH, HIDDEN)


# speedup vs baseline: 2.4692x; 1.9278x over previous
import functools

import jax
import jax.numpy as jnp
from jax import lax
from jax.experimental import pallas as pl
from jax.experimental.pallas import tpu as pltpu
from jax.experimental.pallas import tpu_sc as plsc

HIDDEN = 1024
MAXPOS = 8192
BATCH = 4
SEQ = 8192

_info = plsc.get_sparse_core_info()
NC = _info.num_cores
NS = _info.num_subcores
NW = NC * NS
TOTAL = BATCH * SEQ
B_PER_W = TOTAL // NW
CHUNK = 32
SPC = CHUNK // BATCH
NCHUNK = B_PER_W // CHUNK
NPAIR = NCHUNK // 2

_mesh = plsc.VectorSubcoreMesh(core_axis_name="c", subcore_axis_name="s")


@functools.partial(
    pl.kernel,
    mesh=_mesh,
    out_type=jax.ShapeDtypeStruct((SEQ, BATCH, HIDDEN), jnp.float32),
    scratch_types=[
        pltpu.VMEM((B_PER_W,), jnp.int32),
        pltpu.VMEM((CHUNK, HIDDEN), jnp.float32),
        pltpu.VMEM((CHUNK, HIDDEN), jnp.float32),
        pltpu.SemaphoreType.DMA,
        pltpu.SemaphoreType.DMA,
        pltpu.SemaphoreType.DMA,
        pltpu.SemaphoreType.DMA,
    ],
)
def _emb_lookup(idx_hbm, table_hbm, out_hbm, idx_v, buf0, buf1,
                gsem0, gsem1, wsem0, wsem1):
    wid = lax.axis_index("s") * NC + lax.axis_index("c")
    base = wid * B_PER_W
    sbase = wid * (B_PER_W // BATCH)
    pltpu.sync_copy(idx_hbm.at[pl.ds(base, B_PER_W)], idx_v)

    def gather(chunk, buf, sem):
        return pltpu.make_async_copy(
            table_hbm.at[idx_v.at[pl.ds(chunk * CHUNK, CHUNK)]], buf, sem)

    def group_write(chunk, j, buf, sem):
        s = sbase + chunk * SPC + j
        return pltpu.make_async_copy(
            buf.at[pl.ds(j * BATCH, BATCH)], out_hbm.at[s], sem)

    def fire_writes(chunk, buf, sem):
        for j in range(SPC):
            group_write(chunk, j, buf, sem).start()

    def drain_writes(chunk, buf, sem):
        for j in range(SPC):
            group_write(chunk, j, buf, sem).wait()

    gather(0, buf0, gsem0).start()

    def body(g, carry):
        c0 = 2 * g
        gather(c0, buf0, gsem0).wait()
        fire_writes(c0, buf0, wsem0)
        pl.when(g > 0)(lambda: drain_writes(c0 - 1, buf1, wsem1))
        gather(c0 + 1, buf1, gsem1).start()
        gather(c0 + 1, buf1, gsem1).wait()
        fire_writes(c0 + 1, buf1, wsem1)
        drain_writes(c0, buf0, wsem0)
        pl.when(g < NPAIR - 1)(lambda: gather(c0 + 2, buf0, gsem0).start())
        return carry

    lax.fori_loop(0, NPAIR, body, 0)
    drain_writes(NCHUNK - 1, buf1, wsem1)


def kernel(position_ids, table):
    idx_flat = jnp.swapaxes(position_ids, 0, 1).reshape(TOTAL).astype(jnp.int32)
    return _emb_lookup(idx_flat, table)


# speedup vs baseline: 2.8710x; 1.1628x over previous
import functools

import jax
import jax.numpy as jnp
from jax import lax
from jax.experimental import pallas as pl
from jax.experimental.pallas import tpu as pltpu
from jax.experimental.pallas import tpu_sc as plsc

HIDDEN = 1024
MAXPOS = 8192
BATCH = 4
SEQ = 8192

_info = plsc.get_sparse_core_info()
NC = _info.num_cores
NS = _info.num_subcores
NW = NC * NS
TOTAL = BATCH * SEQ
B_PER_W = TOTAL // NW
NB = 4
CHUNK = 16
SPC = CHUNK // BATCH
NCHUNK = B_PER_W // CHUNK
G = NCHUNK // NB

_mesh = plsc.VectorSubcoreMesh(core_axis_name="c", subcore_axis_name="s")


@functools.partial(
    pl.kernel,
    mesh=_mesh,
    out_type=jax.ShapeDtypeStruct((SEQ, BATCH, HIDDEN), jnp.float32),
    scratch_types=[
        pltpu.VMEM((B_PER_W,), jnp.int32),
        pltpu.VMEM((NB, CHUNK, HIDDEN), jnp.float32),
        pltpu.SemaphoreType.DMA,
        pltpu.SemaphoreType.DMA,
        pltpu.SemaphoreType.DMA,
        pltpu.SemaphoreType.DMA,
        pltpu.SemaphoreType.DMA,
        pltpu.SemaphoreType.DMA,
        pltpu.SemaphoreType.DMA,
        pltpu.SemaphoreType.DMA,
    ],
)
def _emb_lookup(idx_hbm, table_hbm, out_hbm, idx_v, bufs, *sems):
    gsem = sems[:NB]
    wsem = sems[NB:]
    wid = lax.axis_index("s") * NC + lax.axis_index("c")
    base = wid * B_PER_W
    sbase = wid * (B_PER_W // BATCH)
    pltpu.sync_copy(idx_hbm.at[pl.ds(base, B_PER_W)], idx_v)

    def gather(chunk, k):
        return pltpu.make_async_copy(
            table_hbm.at[idx_v.at[pl.ds(chunk * CHUNK, CHUNK)]],
            bufs.at[k], gsem[k])

    def group_write(chunk, j, k):
        s = sbase + chunk * SPC + j
        return pltpu.make_async_copy(
            bufs.at[k].at[pl.ds(j * BATCH, BATCH)], out_hbm.at[s], wsem[k])

    for k in range(NB - 1):
        gather(k, k).start()

    def body(g, carry):
        for k in range(NB):
            c = NB * g + k
            gather(c, k).wait()
            for j in range(SPC):
                group_write(c, j, k).start()
            kp = (k - 1) % NB
            def drain(c=c, kp=kp):
                for j in range(SPC):
                    group_write(c - 1, j, kp).wait()
            def prefetch(c=c, kp=kp):
                gather(c + NB - 1, kp).start()
            if k == 0:
                pl.when(g > 0)(drain)
                prefetch()
            else:
                drain()
                pl.when(g < G - 1)(prefetch)
        return carry

    lax.fori_loop(0, G, body, 0)
    for j in range(SPC):
        group_write(NCHUNK - 1, j, (NCHUNK - 1) % NB).wait()


def kernel(position_ids, table):
    idx_flat = jnp.swapaxes(position_ids, 0, 1).reshape(TOTAL).astype(jnp.int32)
    return _emb_lookup(idx_flat, table)


# speedup vs baseline: 2.9562x; 1.0297x over previous
import functools

import jax
import jax.numpy as jnp
from jax import lax
from jax.experimental import pallas as pl
from jax.experimental.pallas import tpu as pltpu
from jax.experimental.pallas import tpu_sc as plsc

HIDDEN = 1024
MAXPOS = 8192
BATCH = 4
SEQ = 8192

_info = plsc.get_sparse_core_info()
NC = _info.num_cores
NS = _info.num_subcores
NW = NC * NS
SPW = SEQ // NW
SPU = 8
UNITS = BATCH * (SPW // SPU)
NB = 8
G = UNITS // NB

_mesh = plsc.VectorSubcoreMesh(core_axis_name="c", subcore_axis_name="s")


@functools.partial(
    pl.kernel,
    mesh=_mesh,
    out_type=jax.ShapeDtypeStruct((SEQ, BATCH, HIDDEN), jnp.float32),
    scratch_types=[
        pltpu.VMEM((BATCH, SPW), jnp.int32),
        pltpu.VMEM((NB, SPU, HIDDEN), jnp.float32),
    ] + [pltpu.SemaphoreType.DMA] * (2 * NB),
)
def _emb_lookup(idx_hbm, table_hbm, out_hbm, seg_v, bufs, *sems):
    gsem = sems[:NB]
    wsem = sems[NB:]
    wid = lax.axis_index("s") * NC + lax.axis_index("c")
    sbase = wid * SPW

    for b in range(BATCH):
        pltpu.sync_copy(idx_hbm.at[pl.ds(b * SEQ + sbase, SPW)], seg_v.at[b])

    def gather(u, k):
        b = u % BATCH
        c = u // BATCH
        return pltpu.make_async_copy(
            table_hbm.at[seg_v.at[b, pl.ds(c * SPU, SPU)]],
            bufs.at[k], gsem[k])

    def write(u, k):
        b = u % BATCH
        c = u // BATCH
        return pltpu.make_async_copy(
            bufs.at[k], out_hbm.at[pl.ds(sbase + c * SPU, SPU), b], wsem[k])

    for k in range(NB - 1):
        gather(k, k).start()

    def body(g, carry):
        for k in range(NB):
            u = NB * g + k
            gather(u, k).wait()
            write(u, k).start()
            kp = (k - 1) % NB
            def drain(u=u, kp=kp):
                write(u - 1, kp).wait()
            def prefetch(u=u, kp=kp):
                gather(u + NB - 1, kp).start()
            if k == 0:
                pl.when(g > 0)(drain)
                prefetch()
            else:
                drain()
                pl.when(g < G - 1)(prefetch)
        return carry

    lax.fori_loop(0, G, body, 0)
    write(UNITS - 1, (UNITS - 1) % NB).wait()


def kernel(position_ids, table):
    idx_flat = position_ids.reshape(BATCH * SEQ).astype(jnp.int32)
    return _emb_lookup(idx_flat, table)


# speedup vs baseline: 2.9966x; 1.0136x over previous
import functools

import jax
import jax.numpy as jnp
from jax import lax
from jax.experimental import pallas as pl
from jax.experimental.pallas import tpu as pltpu
from jax.experimental.pallas import tpu_sc as plsc

HIDDEN = 1024
MAXPOS = 8192
BATCH = 4
SEQ = 8192

_info = plsc.get_sparse_core_info()
NC = _info.num_cores
NS = _info.num_subcores
NW = NC * NS
SPW = SEQ // NW
SPU = 8
UNITS = BATCH * (SPW // SPU)
NB = 8
G = UNITS // NB

_mesh = plsc.VectorSubcoreMesh(core_axis_name="c", subcore_axis_name="s")


@functools.partial(
    pl.kernel,
    mesh=_mesh,
    out_type=jax.ShapeDtypeStruct((SEQ, BATCH, HIDDEN), jnp.float32),
    scratch_types=[
        pltpu.VMEM((BATCH, SPW), jnp.int32),
        pltpu.VMEM((NB, SPU, HIDDEN), jnp.float32),
    ] + [pltpu.SemaphoreType.DMA] * (2 * NB),
)
def _emb_lookup(idx_hbm, table_hbm, out_hbm, seg_v, bufs, *sems):
    gsem = sems[:NB]
    wsem = sems[NB:]
    wid = lax.axis_index("s") * NC + lax.axis_index("c")
    sbase = wid * SPW

    pltpu.sync_copy(idx_hbm.at[:, pl.ds(sbase, SPW)], seg_v)

    def gather(u, k):
        b = u % BATCH
        c = u // BATCH
        return pltpu.make_async_copy(
            table_hbm.at[seg_v.at[b, pl.ds(c * SPU, SPU)]],
            bufs.at[k], gsem[k])

    def write(u, k):
        b = u % BATCH
        c = u // BATCH
        return pltpu.make_async_copy(
            bufs.at[k], out_hbm.at[pl.ds(sbase + c * SPU, SPU), b], wsem[k])

    for k in range(NB - 1):
        gather(k, k).start()

    def body(g, carry):
        for k in range(NB):
            u = NB * g + k
            gather(u, k).wait()
            write(u, k).start()
            kp = (k - 1) % NB
            def drain(u=u, kp=kp):
                write(u - 1, kp).wait()
            def prefetch(u=u, kp=kp):
                gather(u + NB - 1, kp).start()
            if k == 0:
                pl.when(g > 0)(drain)
                prefetch()
            else:
                drain()
                pl.when(g < G - 1)(prefetch)
        return carry

    lax.fori_loop(0, G, body, 0)
    write(UNITS - 1, (UNITS - 1) % NB).wait()


def kernel(position_ids, table):
    return _emb_lookup(position_ids.astype(jnp.int32), table)
